# Initial kernel scaffold; baseline (speedup 1.0000x reference)
import jax
import jax.numpy as jnp
from jax.experimental import pallas as pl


def kernel(token_ids, lookup_table):
    raise NotImplementedError("write your pallas kernel here")


# speedup vs baseline: 321.8997x; 321.8997x over previous
import jax
import jax.numpy as jnp
from jax.experimental import pallas as pl

_NUM_TAGS = 3
_NUM_VALUES = 8
_NUM_CLASSES = 8
_TAG_START = 46
_TASK_START = 49
_LOGIT_SCALE = 12.0
_ROLES = ((3, 22), (4, 30), (5, 38))


def _body(tok_ref, tbl_ref, out_ref):
    t = tok_ref[...]
    Bb, S = t.shape
    pos = jax.lax.broadcasted_iota(jnp.int32, (Bb, S), 1)
    is_tag = (t >= _TAG_START) & (t < _TAG_START + _NUM_TAGS)
    key = jnp.where(is_tag, pos * 4 + (t - _TAG_START), -1)
    k = 1
    while k < S:
        shifted = jnp.concatenate(
            [jnp.full((Bb, k), -1, jnp.int32), key[:, : S - k]], axis=1
        )
        key = jnp.maximum(key, shifted)
        k *= 2
    nxt = jnp.concatenate([t[:, 1:], jnp.zeros((Bb, 1), jnp.int32)], axis=1)
    qt = jnp.clip(t[:, S - 1 : S] - _TAG_START, 0, _NUM_TAGS - 1)
    task = jnp.clip(t[:, 1:2] - _TASK_START, 0, 3)
    base = (key >= 0) & ((key & 3) == qt)
    idxs = []
    for role, start in _ROLES:
        m = base & (t == role) & (nxt >= start) & (nxt < start + _NUM_VALUES)
        val = jnp.where(m, nxt - start, _NUM_VALUES)
        minv = jnp.min(val, axis=1, keepdims=True)
        idxs.append(minv & 7)
    a, b, c = idxs
    affine = (a + 2 * b + 3 * c) & 7
    gate = (a * (b + 1) + c * ((a ^ b) + 1)) & 7
    sel = jnp.where(task == 0, a, jnp.where(task == 1, affine, gate))
    cls = jax.lax.broadcasted_iota(jnp.int32, (Bb, _NUM_CLASSES), 1)
    onehot_logits = jnp.where(cls == sel, _LOGIT_SCALE, 0.0).astype(jnp.float32)
    flat = a * 64 + b * 8 + c
    f_iota = jax.lax.broadcasted_iota(jnp.int32, (Bb, 512), 1)
    oh = (f_iota == flat).astype(jnp.float32)
    look = jnp.dot(oh, tbl_ref[...], preferred_element_type=jnp.float32)
    out_ref[...] = jnp.where(task == 3, look, onehot_logits)


@jax.jit
def kernel(token_ids, lookup_table):
    B, S = token_ids.shape
    Bb = min(256, B)
    tbl = lookup_table.reshape(512, 8)
    return pl.pallas_call(
        _body,
        grid=(B // Bb,),
        in_specs=[
            pl.BlockSpec((Bb, S), lambda i: (i, 0)),
            pl.BlockSpec((512, 8), lambda i: (0, 0)),
        ],
        out_specs=pl.BlockSpec((Bb, 8), lambda i: (i, 0)),
        out_shape=jax.ShapeDtypeStruct((B, 8), jnp.float32),
    )(token_ids, tbl)
